# baseline (speedup 1.0000x reference)
import functools
import math

import jax
import jax.numpy as jnp
from jax import lax
from jax.experimental import pallas as pl
from jax.experimental.pallas import tpu as pltpu
from jax.experimental.pallas import tpu_sc as plsc

D_MODEL = 64
SCALE = math.sqrt(D_MODEL)

NUM_CORES = 2
NUM_SUBCORES = 16
NW = NUM_CORES * NUM_SUBCORES
LANES = 16

CHUNK = 128
NBUF = 2


def _make_gather(B: int, V: int):
    assert B % (NW * CHUNK * NBUF) == 0
    b_per_w = B // NW
    n_chunks = b_per_w // CHUNK

    mesh = plsc.VectorSubcoreMesh(core_axis_name="c", subcore_axis_name="s")

    @functools.partial(
        pl.kernel,
        mesh=mesh,
        compiler_params=pltpu.CompilerParams(use_tc_tiling_on_sc=False),
        out_type=jax.ShapeDtypeStruct((B, D_MODEL), jnp.float32),
        scratch_types=[
            pltpu.VMEM((n_chunks, CHUNK), jnp.int32),
            pltpu.VMEM((NBUF, CHUNK, D_MODEL), jnp.float32),
            pltpu.SemaphoreType.DMA,
            pltpu.SemaphoreType.DMA,
        ],
    )
    def gather_scale(idx_hbm, w_hbm, out_hbm, idx_v, rows_v, gsem, ssem):
        wid = lax.axis_index("s") * NUM_CORES + lax.axis_index("c")
        base = wid * b_per_w

        pltpu.sync_copy(idx_hbm.at[wid], idx_v)

        def start_gather(g, b):
            pltpu.async_copy(w_hbm.at[idx_v.at[g]], rows_v.at[b], gsem)

        def wait_gather(g, b):
            pltpu.make_async_copy(w_hbm.at[idx_v.at[g]], rows_v.at[b], gsem).wait()

        def wait_store(b):
            pltpu.make_async_copy(
                rows_v.at[b], out_hbm.at[pl.ds(base, CHUNK)], ssem
            ).wait()

        def scale_chunk(b):
            def row_body(r, carry):
                for j in range(D_MODEL // LANES):
                    sl = pl.ds(j * LANES, LANES)
                    rows_v[b, r, sl] = rows_v[b, r, sl] * SCALE
                return carry

            lax.fori_loop(0, CHUNK, row_body, 0, unroll=2)

        start_gather(0, 0)

        def outer(g2, carry):
            g0 = g2 * NBUF
            for b in range(NBUF):
                g = g0 + b
                wait_gather(g, b)
                scale_chunk(b)
                pltpu.async_copy(
                    rows_v.at[b], out_hbm.at[pl.ds(base + g * CHUNK, CHUNK)], ssem
                )

                @pl.when(g >= 1)
                def _():
                    wait_store(1 - b)

                @pl.when(g + 1 < n_chunks)
                def _():
                    start_gather(g + 1, 1 - b)
            return carry

        lax.fori_loop(0, n_chunks // NBUF, outer, 0)
        wait_store((n_chunks - 1) % NBUF)

    return gather_scale


def kernel(x, W):
    B0, F = x.shape
    V, D = W.shape
    B = B0 * F
    idx = x.reshape(NW, B // (NW * CHUNK), CHUNK).astype(jnp.int32)
    out = _make_gather(B, V)(idx, W)
    return out.reshape(B0, F, D)


# speedup vs baseline: 1.0067x; 1.0067x over previous
import functools
import math

import jax
import jax.numpy as jnp
from jax import lax
from jax.experimental import pallas as pl
from jax.experimental.pallas import tpu as pltpu
from jax.experimental.pallas import tpu_sc as plsc

D_MODEL = 64
SCALE = math.sqrt(D_MODEL)

NUM_CORES = 2
NUM_SUBCORES = 16
NW = NUM_CORES * NUM_SUBCORES
LANES = 16

CHUNK = 128
NGBUF = 4
NSBUF = 2


def _make_gather(B: int, V: int):
    assert B % (NW * CHUNK * NGBUF) == 0
    b_per_w = B // NW
    n_chunks = b_per_w // CHUNK

    mesh = plsc.VectorSubcoreMesh(core_axis_name="c", subcore_axis_name="s")

    @functools.partial(
        pl.kernel,
        mesh=mesh,
        compiler_params=pltpu.CompilerParams(use_tc_tiling_on_sc=False),
        out_type=jax.ShapeDtypeStruct((B, D_MODEL), jnp.float32),
        scratch_types=[
            pltpu.VMEM((n_chunks, CHUNK), jnp.int32),
            pltpu.VMEM((NGBUF, CHUNK, D_MODEL), jnp.float32),
            pltpu.VMEM((NSBUF, CHUNK, D_MODEL), jnp.float32),
            pltpu.SemaphoreType.DMA,
            pltpu.SemaphoreType.DMA,
        ],
    )
    def gather_scale(idx_hbm, w_hbm, out_hbm, idx_v, gbuf, sbuf, gsem, ssem):
        wid = lax.axis_index("s") * NUM_CORES + lax.axis_index("c")
        base = wid * b_per_w

        pltpu.sync_copy(idx_hbm.at[wid], idx_v)

        def start_gather(g, b):
            pltpu.async_copy(w_hbm.at[idx_v.at[g]], gbuf.at[b], gsem)

        def wait_gather(g, b):
            pltpu.make_async_copy(w_hbm.at[idx_v.at[g]], gbuf.at[b], gsem).wait()

        def wait_store(b):
            pltpu.make_async_copy(
                sbuf.at[b], out_hbm.at[pl.ds(base, CHUNK)], ssem
            ).wait()

        def scale_chunk(gb, sb):
            def row_body(r, carry):
                for j in range(D_MODEL // LANES):
                    sl = pl.ds(j * LANES, LANES)
                    sbuf[sb, r, sl] = gbuf[gb, r, sl] * SCALE
                return carry

            lax.fori_loop(0, CHUNK, row_body, 0, unroll=4)

        for b in range(NGBUF):
            start_gather(b, b)

        def outer(g2, carry):
            g0 = g2 * NGBUF
            for b in range(NGBUF):
                g = g0 + b
                sb = b % NSBUF
                wait_gather(g, b)

                @pl.when(g >= NSBUF)
                def _():
                    wait_store(sb)

                scale_chunk(b, sb)
                pltpu.async_copy(
                    sbuf.at[sb], out_hbm.at[pl.ds(base + g * CHUNK, CHUNK)], ssem
                )

                @pl.when(g + NGBUF < n_chunks)
                def _():
                    start_gather(g + NGBUF, b)
            return carry

        lax.fori_loop(0, n_chunks // NGBUF, outer, 0)
        for b in range(NSBUF):
            wait_store(b)

    return gather_scale


def kernel(x, W):
    B0, F = x.shape
    V, D = W.shape
    B = B0 * F
    idx = x.reshape(NW, B // (NW * CHUNK), CHUNK).astype(jnp.int32)
    out = _make_gather(B, V)(idx, W)
    return out.reshape(B0, F, D)
